# initial kernel scaffold (unmeasured)
import jax
import jax.numpy as jnp
from jax import lax
from jax.experimental import pallas as pl
from jax.experimental.pallas import tpu as pltpu


def kernel(O, Wo):
    B, S, Hs, D = O.shape
    K = Hs * D
    N = Wo.shape[1]
    M = S // 2
    Oj = O.reshape(B, S, K)

    def body(o_hbm, w_hbm, out_hbm,
             wo_vmem, o_tile, acc, rtile, send_hbm, recv_hbm,
             load_sem, store_sem, send_sem, recv_sem):
        my_x = lax.axis_index("x")
        my_y = lax.axis_index("y")
        nbr = (1 - my_x, my_y)

        barrier = pltpu.get_barrier_semaphore()
        pl.semaphore_signal(barrier, inc=1, device_id=nbr,
                            device_id_type=pl.DeviceIdType.MESH)
        pl.semaphore_wait(barrier, 1)

        cp = pltpu.make_async_copy(w_hbm, wo_vmem, load_sem)
        cp.start()
        cp.wait()

        own0 = my_x * M
        oth0 = (1 - my_x) * M

        def mm(b, row0, dst_hbm):
            ld = pltpu.make_async_copy(
                o_hbm.at[b, pl.ds(row0, M), :], o_tile, load_sem)
            ld.start()
            ld.wait()
            acc[...] = jnp.dot(o_tile[...], wo_vmem[...],
                               preferred_element_type=jnp.float32)
            st = pltpu.make_async_copy(acc, dst_hbm.at[b], store_sem)
            st.start()
            st.wait()

        for b in range(B):
            mm(b, oth0, send_hbm)

        rdma = pltpu.make_async_remote_copy(
            src_ref=send_hbm, dst_ref=recv_hbm,
            send_sem=send_sem, recv_sem=recv_sem,
            device_id=nbr, device_id_type=pl.DeviceIdType.MESH)
        rdma.start()

        for b in range(B):
            mm(b, own0, out_hbm)

        rdma.wait()

        for b in range(B):
            ld1 = pltpu.make_async_copy(out_hbm.at[b], acc, load_sem)
            ld1.start()
            ld1.wait()
            ld2 = pltpu.make_async_copy(recv_hbm.at[b], rtile, load_sem)
            ld2.start()
            ld2.wait()
            acc[...] = acc[...] + rtile[...]
            st = pltpu.make_async_copy(acc, out_hbm.at[b], store_sem)
            st.start()
            st.wait()

    return pl.pallas_call(
        body,
        out_shape=jax.ShapeDtypeStruct((B, M, N), jnp.float32),
        in_specs=[pl.BlockSpec(memory_space=pltpu.MemorySpace.HBM),
                  pl.BlockSpec(memory_space=pltpu.MemorySpace.HBM)],
        out_specs=pl.BlockSpec(memory_space=pltpu.MemorySpace.HBM),
        scratch_shapes=[
            pltpu.VMEM((K, N), jnp.float32),
            pltpu.VMEM((M, K), jnp.float32),
            pltpu.VMEM((M, N), jnp.float32),
            pltpu.VMEM((M, N), jnp.float32),
            pltpu.HBM((B, M, N), jnp.float32),
            pltpu.HBM((B, M, N), jnp.float32),
            pltpu.SemaphoreType.DMA,
            pltpu.SemaphoreType.DMA,
            pltpu.SemaphoreType.DMA,
            pltpu.SemaphoreType.DMA,
        ],
        compiler_params=pltpu.CompilerParams(collective_id=0),
    )(Oj, Wo)


# baseline (device time: 441444 ns/iter reference)
import jax
import jax.numpy as jnp
from jax import lax
from jax.experimental import pallas as pl
from jax.experimental.pallas import tpu as pltpu


def kernel(O, Wo):
    B, S, Hs, D = O.shape
    K = Hs * D
    N = Wo.shape[1]
    NH = N // 2
    M = S // 2
    NCHUNK = 2 * B
    NSLOT = 4
    Oj = O.reshape(B, S, K)

    def body(o_hbm, w_hbm, out_hbm, recv_hbm,
             wo_vmem, o_tile, sbuf, acc, rtile,
             load_sem, store_sem, send_sems, recv_sems):
        my_x = lax.axis_index("x")
        my_y = lax.axis_index("y")
        nbr = (1 - my_x, my_y)

        barrier = pltpu.get_barrier_semaphore()
        pl.semaphore_signal(barrier, inc=1, device_id=nbr,
                            device_id_type=pl.DeviceIdType.MESH)
        pl.semaphore_wait(barrier, 1)

        own0 = my_x * M
        oth0 = (1 - my_x) * M

        def load(src, dst):
            cp = pltpu.make_async_copy(src, dst, load_sem)
            cp.start()
            cp.wait()

        def store(src, dst):
            cp = pltpu.make_async_copy(src, dst, store_sem)
            cp.start()
            cp.wait()

        def chunk_rdma(c):
            return pltpu.make_async_remote_copy(
                src_ref=sbuf.at[c % NSLOT],
                dst_ref=recv_hbm.at[c],
                send_sem=send_sems.at[c],
                recv_sem=recv_sems.at[c],
                device_id=nbr,
                device_id_type=pl.DeviceIdType.MESH)

        for n in range(2):
            load(w_hbm.at[:, pl.ds(n * NH, NH)], wo_vmem)
            for b in range(B):
                c = n * B + b
                load(o_hbm.at[b, pl.ds(oth0, M), :], o_tile)
                if c >= NSLOT:
                    chunk_rdma(c - NSLOT).wait_send()
                sbuf[c % NSLOT] = jnp.dot(o_tile[...], wo_vmem[...],
                                          preferred_element_type=jnp.float32)
                chunk_rdma(c).start()

        for n in range(2):
            load(w_hbm.at[:, pl.ds(n * NH, NH)], wo_vmem)
            for b in range(B):
                load(o_hbm.at[b, pl.ds(own0, M), :], o_tile)
                acc[...] = jnp.dot(o_tile[...], wo_vmem[...],
                                   preferred_element_type=jnp.float32)
                store(acc, out_hbm.at[b, :, pl.ds(n * NH, NH)])

        for n in range(2):
            for b in range(B):
                c = n * B + b
                chunk_rdma(c).wait_recv()
                load(recv_hbm.at[c], rtile)
                load(out_hbm.at[b, :, pl.ds(n * NH, NH)], acc)
                acc[...] = acc[...] + rtile[...]
                store(acc, out_hbm.at[b, :, pl.ds(n * NH, NH)])

        for c in range(NCHUNK - NSLOT, NCHUNK):
            chunk_rdma(c).wait_send()

    out, _ = pl.pallas_call(
        body,
        out_shape=[
            jax.ShapeDtypeStruct((B, M, N), jnp.float32),
            jax.ShapeDtypeStruct((NCHUNK, M, NH), jnp.float32),
        ],
        in_specs=[pl.BlockSpec(memory_space=pltpu.MemorySpace.HBM),
                  pl.BlockSpec(memory_space=pltpu.MemorySpace.HBM)],
        out_specs=[pl.BlockSpec(memory_space=pltpu.MemorySpace.HBM),
                   pl.BlockSpec(memory_space=pltpu.MemorySpace.HBM)],
        scratch_shapes=[
            pltpu.VMEM((K, NH), jnp.float32),
            pltpu.VMEM((M, K), jnp.float32),
            pltpu.VMEM((NSLOT, M, NH), jnp.float32),
            pltpu.VMEM((M, NH), jnp.float32),
            pltpu.VMEM((M, NH), jnp.float32),
            pltpu.SemaphoreType.DMA,
            pltpu.SemaphoreType.DMA,
            pltpu.SemaphoreType.DMA((NCHUNK,)),
            pltpu.SemaphoreType.DMA((NCHUNK,)),
        ],
        compiler_params=pltpu.CompilerParams(
            collective_id=0, vmem_limit_bytes=64 * 1024 * 1024),
    )(Oj, Wo)
    return out


# device time: 231406 ns/iter; 1.9077x vs baseline; 1.9077x over previous
import jax
import jax.numpy as jnp
from jax import lax
from jax.experimental import pallas as pl
from jax.experimental.pallas import tpu as pltpu


def kernel(O, Wo):
    B, S, Hs, D = O.shape
    K = Hs * D
    N = Wo.shape[1]
    NH = N // 2
    M = S // 2
    NSLOT = 2
    Oj = O.reshape(B, S, K)

    def body(o_hbm, w_hbm, out_hbm, xrecv_hbm, yrecv_hbm,
             wo_vmem, o_tile, sbuf, acc, rtile, ybuf,
             load_sem, store_sem,
             xsend_sems, xrecv_sems, ysend_sems, yrecv_sems):
        my_x = lax.axis_index("x")
        my_y = lax.axis_index("y")
        xnbr = (1 - my_x, my_y)
        ynbr = (my_x, 1 - my_y)

        barrier = pltpu.get_barrier_semaphore()
        for nbr in (xnbr, ynbr):
            pl.semaphore_signal(barrier, inc=1, device_id=nbr,
                                device_id_type=pl.DeviceIdType.MESH)
        pl.semaphore_wait(barrier, 2)

        own0 = my_x * M
        oth0 = (1 - my_x) * M
        myn0 = my_y * NH
        othn0 = (1 - my_y) * NH

        def load(src, dst):
            cp = pltpu.make_async_copy(src, dst, load_sem)
            cp.start()
            cp.wait()

        def x_rdma(b):
            return pltpu.make_async_remote_copy(
                src_ref=sbuf.at[b % NSLOT],
                dst_ref=xrecv_hbm.at[b],
                send_sem=xsend_sems.at[b],
                recv_sem=xrecv_sems.at[b],
                device_id=xnbr,
                device_id_type=pl.DeviceIdType.MESH)

        def y_rdma(b):
            return pltpu.make_async_remote_copy(
                src_ref=ybuf.at[b],
                dst_ref=yrecv_hbm.at[b],
                send_sem=ysend_sems.at[b],
                recv_sem=yrecv_sems.at[b],
                device_id=ynbr,
                device_id_type=pl.DeviceIdType.MESH)

        load(w_hbm.at[:, pl.ds(myn0, NH)], wo_vmem)

        for b in range(B):
            load(o_hbm.at[b, pl.ds(oth0, M), :], o_tile)
            if b >= NSLOT:
                x_rdma(b - NSLOT).wait_send()
            sbuf[b % NSLOT] = jnp.dot(
                o_tile[...], wo_vmem[...],
                preferred_element_type=jnp.float32).astype(jnp.bfloat16)
            x_rdma(b).start()

        for b in range(B):
            load(o_hbm.at[b, pl.ds(own0, M), :], o_tile)
            acc[...] = jnp.dot(o_tile[...], wo_vmem[...],
                               preferred_element_type=jnp.float32)
            x_rdma(b).wait_recv()
            load(xrecv_hbm.at[b], rtile)
            acc[...] = acc[...] + rtile[...].astype(jnp.float32)
            ybuf[b] = acc[...].astype(jnp.bfloat16)
            st = pltpu.make_async_copy(
                acc, out_hbm.at[b, :, pl.ds(myn0, NH)], store_sem)
            st.start()
            st.wait()
            y_rdma(b).start()

        for b in range(B):
            y_rdma(b).wait_recv()
            load(yrecv_hbm.at[b], rtile)
            acc[...] = rtile[...].astype(jnp.float32)
            st = pltpu.make_async_copy(
                acc, out_hbm.at[b, :, pl.ds(othn0, NH)], store_sem)
            st.start()
            st.wait()
        for b in range(max(0, B - NSLOT), B):
            x_rdma(b).wait_send()
        for b in range(B):
            y_rdma(b).wait_send()

    out, _, _ = pl.pallas_call(
        body,
        out_shape=[
            jax.ShapeDtypeStruct((B, M, N), jnp.float32),
            jax.ShapeDtypeStruct((B, M, NH), jnp.bfloat16),
            jax.ShapeDtypeStruct((B, M, NH), jnp.bfloat16),
        ],
        in_specs=[pl.BlockSpec(memory_space=pltpu.MemorySpace.HBM),
                  pl.BlockSpec(memory_space=pltpu.MemorySpace.HBM)],
        out_specs=[pl.BlockSpec(memory_space=pltpu.MemorySpace.HBM),
                   pl.BlockSpec(memory_space=pltpu.MemorySpace.HBM),
                   pl.BlockSpec(memory_space=pltpu.MemorySpace.HBM)],
        scratch_shapes=[
            pltpu.VMEM((K, NH), jnp.float32),
            pltpu.VMEM((M, K), jnp.float32),
            pltpu.VMEM((NSLOT, M, NH), jnp.bfloat16),
            pltpu.VMEM((M, NH), jnp.float32),
            pltpu.VMEM((M, NH), jnp.bfloat16),
            pltpu.VMEM((B, M, NH), jnp.bfloat16),
            pltpu.SemaphoreType.DMA,
            pltpu.SemaphoreType.DMA,
            pltpu.SemaphoreType.DMA((B,)),
            pltpu.SemaphoreType.DMA((B,)),
            pltpu.SemaphoreType.DMA((B,)),
            pltpu.SemaphoreType.DMA((B,)),
        ],
        compiler_params=pltpu.CompilerParams(
            collective_id=0, vmem_limit_bytes=64 * 1024 * 1024),
    )(Oj, Wo)
    return out


# device time: 221081 ns/iter; 1.9968x vs baseline; 1.0467x over previous
import jax
import jax.numpy as jnp
from jax import lax
from jax.experimental import pallas as pl
from jax.experimental.pallas import tpu as pltpu


def kernel(O, Wo):
    B, S, Hs, D = O.shape
    K = Hs * D
    N = Wo.shape[1]
    NH = N // 2
    M = S // 2
    SPLIT = 2
    MC = M // SPLIT
    C = B * SPLIT
    NSLOT = 4
    Oj = O.reshape(B, S, K)

    def body(o_hbm, w_hbm, out_hbm, xrecv_hbm, yrecv_hbm,
             wo_vmem, o_tile, sbuf, acc, rtile, ybuf,
             load_sem, store_sem,
             xsend_sems, xrecv_sems, ysend_sems, yrecv_sems):
        my_x = lax.axis_index("x")
        my_y = lax.axis_index("y")
        xnbr = (1 - my_x, my_y)
        ynbr = (my_x, 1 - my_y)

        barrier = pltpu.get_barrier_semaphore()
        for nbr in (xnbr, ynbr):
            pl.semaphore_signal(barrier, inc=1, device_id=nbr,
                                device_id_type=pl.DeviceIdType.MESH)
        pl.semaphore_wait(barrier, 2)

        own0 = my_x * M
        oth0 = (1 - my_x) * M
        myn0 = my_y * NH
        othn0 = (1 - my_y) * NH

        def load(src, dst):
            cp = pltpu.make_async_copy(src, dst, load_sem)
            cp.start()
            cp.wait()

        def x_rdma(c):
            return pltpu.make_async_remote_copy(
                src_ref=sbuf.at[c % NSLOT],
                dst_ref=xrecv_hbm.at[c],
                send_sem=xsend_sems.at[c],
                recv_sem=xrecv_sems.at[c],
                device_id=xnbr,
                device_id_type=pl.DeviceIdType.MESH)

        def y_rdma(c):
            return pltpu.make_async_remote_copy(
                src_ref=ybuf.at[c],
                dst_ref=yrecv_hbm.at[c],
                send_sem=ysend_sems.at[c],
                recv_sem=yrecv_sems.at[c],
                device_id=ynbr,
                device_id_type=pl.DeviceIdType.MESH)

        load(w_hbm.at[:, pl.ds(myn0, NH)], wo_vmem)

        def rows(c, base):
            b, s = divmod(c, SPLIT)
            return b, base + s * MC

        for c in range(C):
            b, r0 = rows(c, oth0)
            load(o_hbm.at[b, pl.ds(r0, MC), :], o_tile)
            if c >= NSLOT:
                x_rdma(c - NSLOT).wait_send()
            sbuf[c % NSLOT] = jnp.dot(
                o_tile[...], wo_vmem[...],
                preferred_element_type=jnp.float32).astype(jnp.bfloat16)
            x_rdma(c).start()

        for c in range(C):
            b, r0 = rows(c, own0)
            load(o_hbm.at[b, pl.ds(r0, MC), :], o_tile)
            acc[...] = jnp.dot(o_tile[...], wo_vmem[...],
                               preferred_element_type=jnp.float32)
            x_rdma(c).wait_recv()
            load(xrecv_hbm.at[c], rtile)
            acc[...] = acc[...] + rtile[...].astype(jnp.float32)
            ybuf[c] = acc[...].astype(jnp.bfloat16)
            st = pltpu.make_async_copy(
                acc, out_hbm.at[b, pl.ds((c % SPLIT) * MC, MC),
                                pl.ds(myn0, NH)], store_sem)
            st.start()
            st.wait()
            y_rdma(c).start()

        for c in range(C):
            b, _ = rows(c, 0)
            y_rdma(c).wait_recv()
            load(yrecv_hbm.at[c], rtile)
            acc[...] = rtile[...].astype(jnp.float32)
            st = pltpu.make_async_copy(
                acc, out_hbm.at[b, pl.ds((c % SPLIT) * MC, MC),
                                pl.ds(othn0, NH)], store_sem)
            st.start()
            st.wait()
        for c in range(max(0, C - NSLOT), C):
            x_rdma(c).wait_send()
        for c in range(C):
            y_rdma(c).wait_send()

    out, _, _ = pl.pallas_call(
        body,
        out_shape=[
            jax.ShapeDtypeStruct((B, M, N), jnp.float32),
            jax.ShapeDtypeStruct((C, MC, NH), jnp.bfloat16),
            jax.ShapeDtypeStruct((C, MC, NH), jnp.bfloat16),
        ],
        in_specs=[pl.BlockSpec(memory_space=pltpu.MemorySpace.HBM),
                  pl.BlockSpec(memory_space=pltpu.MemorySpace.HBM)],
        out_specs=[pl.BlockSpec(memory_space=pltpu.MemorySpace.HBM),
                   pl.BlockSpec(memory_space=pltpu.MemorySpace.HBM),
                   pl.BlockSpec(memory_space=pltpu.MemorySpace.HBM)],
        scratch_shapes=[
            pltpu.VMEM((K, NH), jnp.float32),
            pltpu.VMEM((MC, K), jnp.float32),
            pltpu.VMEM((NSLOT, MC, NH), jnp.bfloat16),
            pltpu.VMEM((MC, NH), jnp.float32),
            pltpu.VMEM((MC, NH), jnp.bfloat16),
            pltpu.VMEM((C, MC, NH), jnp.bfloat16),
            pltpu.SemaphoreType.DMA,
            pltpu.SemaphoreType.DMA,
            pltpu.SemaphoreType.DMA((C,)),
            pltpu.SemaphoreType.DMA((C,)),
            pltpu.SemaphoreType.DMA((C,)),
            pltpu.SemaphoreType.DMA((C,)),
        ],
        compiler_params=pltpu.CompilerParams(
            collective_id=0, vmem_limit_bytes=64 * 1024 * 1024),
    )(Oj, Wo)
    return out


# device time: 198018 ns/iter; 2.2293x vs baseline; 1.1165x over previous
import jax
import jax.numpy as jnp
from jax import lax
from jax.experimental import pallas as pl
from jax.experimental.pallas import tpu as pltpu


def kernel(O, Wo):
    B, S, Hs, D = O.shape
    K = Hs * D
    N = Wo.shape[1]
    NH = N // 2
    M = S // 2
    SPLIT = 2
    MC = M // SPLIT
    C = B * SPLIT
    NSLOT = 4
    Oj = O.reshape(B, S, K)

    def body(o_hbm, w_hbm, out_hbm, xrecv_hbm, yrecv_hbm,
             wo_vmem, o_tile, sbuf, acc, rtile, rtile8, ybuf,
             load_sem, store_sem,
             xsend_sems, xrecv_sems, ysend_sems, yrecv_sems):
        my_x = lax.axis_index("x")
        my_y = lax.axis_index("y")
        xnbr = (1 - my_x, my_y)
        ynbr = (my_x, 1 - my_y)

        barrier = pltpu.get_barrier_semaphore()
        for nbr in (xnbr, ynbr):
            pl.semaphore_signal(barrier, inc=1, device_id=nbr,
                                device_id_type=pl.DeviceIdType.MESH)
        pl.semaphore_wait(barrier, 2)

        own0 = my_x * M
        oth0 = (1 - my_x) * M
        myn0 = my_y * NH
        othn0 = (1 - my_y) * NH

        def load(src, dst):
            cp = pltpu.make_async_copy(src, dst, load_sem)
            cp.start()
            cp.wait()

        def x_rdma(c):
            return pltpu.make_async_remote_copy(
                src_ref=sbuf.at[c % NSLOT],
                dst_ref=xrecv_hbm.at[c],
                send_sem=xsend_sems.at[c],
                recv_sem=xrecv_sems.at[c],
                device_id=xnbr,
                device_id_type=pl.DeviceIdType.MESH)

        def y_rdma(c):
            return pltpu.make_async_remote_copy(
                src_ref=ybuf.at[c],
                dst_ref=yrecv_hbm.at[c],
                send_sem=ysend_sems.at[c],
                recv_sem=yrecv_sems.at[c],
                device_id=ynbr,
                device_id_type=pl.DeviceIdType.MESH)

        load(w_hbm.at[:, pl.ds(myn0, NH)], wo_vmem)

        def rows(c, base):
            b, s = divmod(c, SPLIT)
            return b, base + s * MC

        for c in range(C):
            b, r0 = rows(c, oth0)
            load(o_hbm.at[b, pl.ds(r0, MC), :], o_tile)
            if c >= NSLOT:
                x_rdma(c - NSLOT).wait_send()
            sbuf[c % NSLOT] = jnp.dot(
                o_tile[...], wo_vmem[...],
                preferred_element_type=jnp.float32).astype(jnp.bfloat16)
            x_rdma(c).start()

        for c in range(C):
            b, r0 = rows(c, own0)
            load(o_hbm.at[b, pl.ds(r0, MC), :], o_tile)
            acc[...] = jnp.dot(o_tile[...], wo_vmem[...],
                               preferred_element_type=jnp.float32)
            x_rdma(c).wait_recv()
            load(xrecv_hbm.at[c], rtile)
            acc[...] = acc[...] + rtile[...].astype(jnp.float32)
            ybuf[c] = acc[...].astype(jnp.float8_e4m3fn)
            st = pltpu.make_async_copy(
                acc, out_hbm.at[b, pl.ds((c % SPLIT) * MC, MC),
                                pl.ds(myn0, NH)], store_sem)
            st.start()
            st.wait()
            y_rdma(c).start()

        for c in range(C):
            b, _ = rows(c, 0)
            y_rdma(c).wait_recv()
            load(yrecv_hbm.at[c], rtile8)
            acc[...] = rtile8[...].astype(jnp.float32)
            st = pltpu.make_async_copy(
                acc, out_hbm.at[b, pl.ds((c % SPLIT) * MC, MC),
                                pl.ds(othn0, NH)], store_sem)
            st.start()
            st.wait()
        for c in range(max(0, C - NSLOT), C):
            x_rdma(c).wait_send()
        for c in range(C):
            y_rdma(c).wait_send()

    out, _, _ = pl.pallas_call(
        body,
        out_shape=[
            jax.ShapeDtypeStruct((B, M, N), jnp.float32),
            jax.ShapeDtypeStruct((C, MC, NH), jnp.bfloat16),
            jax.ShapeDtypeStruct((C, MC, NH), jnp.float8_e4m3fn),
        ],
        in_specs=[pl.BlockSpec(memory_space=pltpu.MemorySpace.HBM),
                  pl.BlockSpec(memory_space=pltpu.MemorySpace.HBM)],
        out_specs=[pl.BlockSpec(memory_space=pltpu.MemorySpace.HBM),
                   pl.BlockSpec(memory_space=pltpu.MemorySpace.HBM),
                   pl.BlockSpec(memory_space=pltpu.MemorySpace.HBM)],
        scratch_shapes=[
            pltpu.VMEM((K, NH), jnp.float32),
            pltpu.VMEM((MC, K), jnp.float32),
            pltpu.VMEM((NSLOT, MC, NH), jnp.bfloat16),
            pltpu.VMEM((MC, NH), jnp.float32),
            pltpu.VMEM((MC, NH), jnp.bfloat16),
            pltpu.VMEM((MC, NH), jnp.float8_e4m3fn),
            pltpu.VMEM((C, MC, NH), jnp.float8_e4m3fn),
            pltpu.SemaphoreType.DMA,
            pltpu.SemaphoreType.DMA,
            pltpu.SemaphoreType.DMA((C,)),
            pltpu.SemaphoreType.DMA((C,)),
            pltpu.SemaphoreType.DMA((C,)),
            pltpu.SemaphoreType.DMA((C,)),
        ],
        compiler_params=pltpu.CompilerParams(
            collective_id=0, vmem_limit_bytes=64 * 1024 * 1024),
    )(Oj, Wo)
    return out


# device time: 184948 ns/iter; 2.3869x vs baseline; 1.0707x over previous
import jax
import jax.numpy as jnp
from jax import lax
from jax.experimental import pallas as pl
from jax.experimental.pallas import tpu as pltpu


def kernel(O, Wo):
    B, S, Hs, D = O.shape
    K = Hs * D
    N = Wo.shape[1]
    NH = N // 2
    M = S // 2
    SPLIT = 1
    MC = M // SPLIT
    C = B * SPLIT
    NSLOT = 4
    Oj = O.reshape(B, S, K)

    def body(o_hbm, w_hbm, out_hbm, xrecv_hbm, yrecv_hbm,
             wo_vmem, o_tile, sbuf, acc, rtile, rtile8, ybuf,
             load_sem, store_sem,
             xsend_sems, xrecv_sems, ysend_sems, yrecv_sems):
        my_x = lax.axis_index("x")
        my_y = lax.axis_index("y")
        xnbr = (1 - my_x, my_y)
        ynbr = (my_x, 1 - my_y)

        barrier = pltpu.get_barrier_semaphore()
        for nbr in (xnbr, ynbr):
            pl.semaphore_signal(barrier, inc=1, device_id=nbr,
                                device_id_type=pl.DeviceIdType.MESH)
        pl.semaphore_wait(barrier, 2)

        own0 = my_x * M
        oth0 = (1 - my_x) * M
        myn0 = my_y * NH
        othn0 = (1 - my_y) * NH

        def load(src, dst):
            cp = pltpu.make_async_copy(src, dst, load_sem)
            cp.start()
            cp.wait()

        def x_rdma(c):
            return pltpu.make_async_remote_copy(
                src_ref=sbuf.at[c % NSLOT],
                dst_ref=xrecv_hbm.at[c],
                send_sem=xsend_sems.at[c],
                recv_sem=xrecv_sems.at[c],
                device_id=xnbr,
                device_id_type=pl.DeviceIdType.MESH)

        def y_rdma(c):
            return pltpu.make_async_remote_copy(
                src_ref=ybuf.at[c],
                dst_ref=yrecv_hbm.at[c],
                send_sem=ysend_sems.at[c],
                recv_sem=yrecv_sems.at[c],
                device_id=ynbr,
                device_id_type=pl.DeviceIdType.MESH)

        load(w_hbm.at[:, pl.ds(myn0, NH)], wo_vmem)

        def rows(c, base):
            b, s = divmod(c, SPLIT)
            return b, base + s * MC

        for c in range(C):
            b, r0 = rows(c, oth0)
            load(o_hbm.at[b, pl.ds(r0, MC), :], o_tile)
            if c >= NSLOT:
                x_rdma(c - NSLOT).wait_send()
            sbuf[c % NSLOT] = jnp.dot(
                o_tile[...], wo_vmem[...],
                preferred_element_type=jnp.float32).astype(jnp.bfloat16)
            x_rdma(c).start()

        for c in range(C):
            b, r0 = rows(c, own0)
            load(o_hbm.at[b, pl.ds(r0, MC), :], o_tile)
            acc[...] = jnp.dot(o_tile[...], wo_vmem[...],
                               preferred_element_type=jnp.float32)
            x_rdma(c).wait_recv()
            load(xrecv_hbm.at[c], rtile)
            acc[...] = acc[...] + rtile[...].astype(jnp.float32)
            ybuf[c] = acc[...].astype(jnp.float8_e4m3fn)
            y_rdma(c).start()
            st = pltpu.make_async_copy(
                acc, out_hbm.at[b, pl.ds((c % SPLIT) * MC, MC),
                                pl.ds(myn0, NH)], store_sem)
            st.start()
            st.wait()

        for c in range(C):
            b, _ = rows(c, 0)
            y_rdma(c).wait_recv()
            load(yrecv_hbm.at[c], rtile8)
            acc[...] = rtile8[...].astype(jnp.float32)
            st = pltpu.make_async_copy(
                acc, out_hbm.at[b, pl.ds((c % SPLIT) * MC, MC),
                                pl.ds(othn0, NH)], store_sem)
            st.start()
            st.wait()
        for c in range(max(0, C - NSLOT), C):
            x_rdma(c).wait_send()
        for c in range(C):
            y_rdma(c).wait_send()

    out, _, _ = pl.pallas_call(
        body,
        out_shape=[
            jax.ShapeDtypeStruct((B, M, N), jnp.float32),
            jax.ShapeDtypeStruct((C, MC, NH), jnp.bfloat16),
            jax.ShapeDtypeStruct((C, MC, NH), jnp.float8_e4m3fn),
        ],
        in_specs=[pl.BlockSpec(memory_space=pltpu.MemorySpace.HBM),
                  pl.BlockSpec(memory_space=pltpu.MemorySpace.HBM)],
        out_specs=[pl.BlockSpec(memory_space=pltpu.MemorySpace.HBM),
                   pl.BlockSpec(memory_space=pltpu.MemorySpace.HBM),
                   pl.BlockSpec(memory_space=pltpu.MemorySpace.HBM)],
        scratch_shapes=[
            pltpu.VMEM((K, NH), jnp.float32),
            pltpu.VMEM((MC, K), jnp.float32),
            pltpu.VMEM((NSLOT, MC, NH), jnp.bfloat16),
            pltpu.VMEM((MC, NH), jnp.float32),
            pltpu.VMEM((MC, NH), jnp.bfloat16),
            pltpu.VMEM((MC, NH), jnp.float8_e4m3fn),
            pltpu.VMEM((C, MC, NH), jnp.float8_e4m3fn),
            pltpu.SemaphoreType.DMA,
            pltpu.SemaphoreType.DMA,
            pltpu.SemaphoreType.DMA((C,)),
            pltpu.SemaphoreType.DMA((C,)),
            pltpu.SemaphoreType.DMA((C,)),
            pltpu.SemaphoreType.DMA((C,)),
        ],
        compiler_params=pltpu.CompilerParams(
            collective_id=0, vmem_limit_bytes=64 * 1024 * 1024),
    )(Oj, Wo)
    return out


# device time: 183794 ns/iter; 2.4018x vs baseline; 1.0063x over previous
import jax
import jax.numpy as jnp
from jax import lax
from jax.experimental import pallas as pl
from jax.experimental.pallas import tpu as pltpu


def kernel(O, Wo):
    B, S, Hs, D = O.shape
    K = Hs * D
    N = Wo.shape[1]
    NH = N // 2
    M = S // 2
    SPLIT = 1
    MC = M // SPLIT
    C = B * SPLIT
    NSLOT = 4
    Oj = O.reshape(B, S, K)

    def body(o_hbm, w_hbm, out_hbm, xrecv_hbm, yrecv_hbm,
             wo_vmem, o_tile, sbuf, acc, rtile, rtile8, ybuf,
             load_sem, store_sem,
             xsend_sems, xrecv_sems, ysend_sems, yrecv_sems):
        my_x = lax.axis_index("x")
        my_y = lax.axis_index("y")
        xnbr = (1 - my_x, my_y)
        ynbr = (my_x, 1 - my_y)

        barrier = pltpu.get_barrier_semaphore()
        for nbr in (xnbr, ynbr):
            pl.semaphore_signal(barrier, inc=1, device_id=nbr,
                                device_id_type=pl.DeviceIdType.MESH)

        own0 = my_x * M
        oth0 = (1 - my_x) * M
        myn0 = my_y * NH
        othn0 = (1 - my_y) * NH

        def load(src, dst):
            cp = pltpu.make_async_copy(src, dst, load_sem)
            cp.start()
            cp.wait()

        def x_rdma(c):
            return pltpu.make_async_remote_copy(
                src_ref=sbuf.at[c % NSLOT],
                dst_ref=xrecv_hbm.at[c],
                send_sem=xsend_sems.at[c],
                recv_sem=xrecv_sems.at[c],
                device_id=xnbr,
                device_id_type=pl.DeviceIdType.MESH)

        def y_rdma(c):
            return pltpu.make_async_remote_copy(
                src_ref=ybuf.at[c],
                dst_ref=yrecv_hbm.at[c],
                send_sem=ysend_sems.at[c],
                recv_sem=yrecv_sems.at[c],
                device_id=ynbr,
                device_id_type=pl.DeviceIdType.MESH)

        load(w_hbm.at[:, pl.ds(myn0, NH)], wo_vmem)

        def rows(c, base):
            b, s = divmod(c, SPLIT)
            return b, base + s * MC

        for c in range(C):
            b, r0 = rows(c, oth0)
            load(o_hbm.at[b, pl.ds(r0, MC), :], o_tile)
            if c >= NSLOT:
                x_rdma(c - NSLOT).wait_send()
            sbuf[c % NSLOT] = jnp.dot(
                o_tile[...], wo_vmem[...],
                preferred_element_type=jnp.float32).astype(jnp.bfloat16)
            if c == 0:
                pl.semaphore_wait(barrier, 2)
            x_rdma(c).start()

        for c in range(C):
            b, r0 = rows(c, own0)
            load(o_hbm.at[b, pl.ds(r0, MC), :], o_tile)
            acc[...] = jnp.dot(o_tile[...], wo_vmem[...],
                               preferred_element_type=jnp.float32)
            x_rdma(c).wait_recv()
            load(xrecv_hbm.at[c], rtile)
            acc[...] = acc[...] + rtile[...].astype(jnp.float32)
            ybuf[c] = acc[...].astype(jnp.float8_e4m3fn)
            y_rdma(c).start()
            st = pltpu.make_async_copy(
                acc, out_hbm.at[b, pl.ds((c % SPLIT) * MC, MC),
                                pl.ds(myn0, NH)], store_sem)
            st.start()
            st.wait()

        for c in range(C):
            b, _ = rows(c, 0)
            y_rdma(c).wait_recv()
            load(yrecv_hbm.at[c], rtile8)
            acc[...] = rtile8[...].astype(jnp.float32)
            st = pltpu.make_async_copy(
                acc, out_hbm.at[b, pl.ds((c % SPLIT) * MC, MC),
                                pl.ds(othn0, NH)], store_sem)
            st.start()
            st.wait()
        for c in range(max(0, C - NSLOT), C):
            x_rdma(c).wait_send()
        for c in range(C):
            y_rdma(c).wait_send()

    out, _, _ = pl.pallas_call(
        body,
        out_shape=[
            jax.ShapeDtypeStruct((B, M, N), jnp.float32),
            jax.ShapeDtypeStruct((C, MC, NH), jnp.bfloat16),
            jax.ShapeDtypeStruct((C, MC, NH), jnp.float8_e4m3fn),
        ],
        in_specs=[pl.BlockSpec(memory_space=pltpu.MemorySpace.HBM),
                  pl.BlockSpec(memory_space=pltpu.MemorySpace.HBM)],
        out_specs=[pl.BlockSpec(memory_space=pltpu.MemorySpace.HBM),
                   pl.BlockSpec(memory_space=pltpu.MemorySpace.HBM),
                   pl.BlockSpec(memory_space=pltpu.MemorySpace.HBM)],
        scratch_shapes=[
            pltpu.VMEM((K, NH), jnp.float32),
            pltpu.VMEM((MC, K), jnp.float32),
            pltpu.VMEM((NSLOT, MC, NH), jnp.bfloat16),
            pltpu.VMEM((MC, NH), jnp.float32),
            pltpu.VMEM((MC, NH), jnp.bfloat16),
            pltpu.VMEM((MC, NH), jnp.float8_e4m3fn),
            pltpu.VMEM((C, MC, NH), jnp.float8_e4m3fn),
            pltpu.SemaphoreType.DMA,
            pltpu.SemaphoreType.DMA,
            pltpu.SemaphoreType.DMA((C,)),
            pltpu.SemaphoreType.DMA((C,)),
            pltpu.SemaphoreType.DMA((C,)),
            pltpu.SemaphoreType.DMA((C,)),
        ],
        compiler_params=pltpu.CompilerParams(
            collective_id=0, vmem_limit_bytes=64 * 1024 * 1024),
    )(Oj, Wo)
    return out


# device time: 165301 ns/iter; 2.6705x vs baseline; 1.1119x over previous
import jax
import jax.numpy as jnp
from jax import lax
from jax.experimental import pallas as pl
from jax.experimental.pallas import tpu as pltpu


def kernel(O, Wo):
    B, S, Hs, D = O.shape
    K = Hs * D
    N = Wo.shape[1]
    NH = N // 2
    M = S // 2
    MC = M
    C = B
    NSLOT = 4
    W = MC + 8
    Oj = O.reshape(B, S, K)

    def body(o_hbm, w_hbm, out_hbm, xrecv_hbm, yrecv_hbm,
             wo_vmem, o_tile, sbuf, acc, rtile, ybuf,
             load_sem, store_sem,
             xsend_sems, xrecv_sems, ysend_sems, yrecv_sems):
        my_x = lax.axis_index("x")
        my_y = lax.axis_index("y")
        xnbr = (1 - my_x, my_y)
        ynbr = (my_x, 1 - my_y)

        barrier = pltpu.get_barrier_semaphore()
        for nbr in (xnbr, ynbr):
            pl.semaphore_signal(barrier, inc=1, device_id=nbr,
                                device_id_type=pl.DeviceIdType.MESH)

        own0 = my_x * M
        oth0 = (1 - my_x) * M
        myn0 = my_y * NH
        othn0 = (1 - my_y) * NH

        def load(src, dst):
            cp = pltpu.make_async_copy(src, dst, load_sem)
            cp.start()
            cp.wait()

        def quantize(p, wire_ref):
            m = jnp.maximum(jnp.max(jnp.abs(p)), 1e-30)
            e = jnp.ceil(8.0 * jnp.log2(m / 126.0))
            wire_ref[:MC, :] = jnp.round(p * jnp.exp2(-e / 8.0)).astype(jnp.int8)
            wire_ref[MC:, :] = jnp.full((8, NH), e, jnp.float32).astype(jnp.int8)

        def dequantize(wire_ref):
            inv = jnp.exp2(wire_ref[MC:MC + 1, :].astype(jnp.float32) / 8.0)
            return wire_ref[:MC, :].astype(jnp.float32) * inv

        def x_rdma(c):
            return pltpu.make_async_remote_copy(
                src_ref=sbuf.at[c % NSLOT],
                dst_ref=xrecv_hbm.at[c],
                send_sem=xsend_sems.at[c],
                recv_sem=xrecv_sems.at[c],
                device_id=xnbr,
                device_id_type=pl.DeviceIdType.MESH)

        def y_rdma(c):
            return pltpu.make_async_remote_copy(
                src_ref=ybuf.at[c],
                dst_ref=yrecv_hbm.at[c],
                send_sem=ysend_sems.at[c],
                recv_sem=yrecv_sems.at[c],
                device_id=ynbr,
                device_id_type=pl.DeviceIdType.MESH)

        load(w_hbm.at[:, pl.ds(myn0, NH)], wo_vmem)

        for c in range(C):
            load(o_hbm.at[c, pl.ds(oth0, MC), :], o_tile)
            if c >= NSLOT:
                x_rdma(c - NSLOT).wait_send()
            p = jnp.dot(o_tile[...], wo_vmem[...],
                        preferred_element_type=jnp.float32)
            quantize(p, sbuf.at[c % NSLOT])
            if c == 0:
                pl.semaphore_wait(barrier, 2)
            x_rdma(c).start()

        for c in range(C):
            load(o_hbm.at[c, pl.ds(own0, MC), :], o_tile)
            acc[...] = jnp.dot(o_tile[...], wo_vmem[...],
                               preferred_element_type=jnp.float32)
            x_rdma(c).wait_recv()
            load(xrecv_hbm.at[c], rtile)
            acc[...] = acc[...] + dequantize(rtile)
            quantize(acc[...], ybuf.at[c])
            y_rdma(c).start()
            st = pltpu.make_async_copy(
                acc, out_hbm.at[c, :, pl.ds(myn0, NH)], store_sem)
            st.start()
            st.wait()

        for c in range(C):
            y_rdma(c).wait_recv()
            load(yrecv_hbm.at[c], rtile)
            acc[...] = dequantize(rtile)
            st = pltpu.make_async_copy(
                acc, out_hbm.at[c, :, pl.ds(othn0, NH)], store_sem)
            st.start()
            st.wait()
        for c in range(max(0, C - NSLOT), C):
            x_rdma(c).wait_send()
        for c in range(C):
            y_rdma(c).wait_send()

    out, _, _ = pl.pallas_call(
        body,
        out_shape=[
            jax.ShapeDtypeStruct((B, M, N), jnp.float32),
            jax.ShapeDtypeStruct((C, W, NH), jnp.int8),
            jax.ShapeDtypeStruct((C, W, NH), jnp.int8),
        ],
        in_specs=[pl.BlockSpec(memory_space=pltpu.MemorySpace.HBM),
                  pl.BlockSpec(memory_space=pltpu.MemorySpace.HBM)],
        out_specs=[pl.BlockSpec(memory_space=pltpu.MemorySpace.HBM),
                   pl.BlockSpec(memory_space=pltpu.MemorySpace.HBM),
                   pl.BlockSpec(memory_space=pltpu.MemorySpace.HBM)],
        scratch_shapes=[
            pltpu.VMEM((K, NH), jnp.float32),
            pltpu.VMEM((MC, K), jnp.float32),
            pltpu.VMEM((NSLOT, W, NH), jnp.int8),
            pltpu.VMEM((MC, NH), jnp.float32),
            pltpu.VMEM((W, NH), jnp.int8),
            pltpu.VMEM((C, W, NH), jnp.int8),
            pltpu.SemaphoreType.DMA,
            pltpu.SemaphoreType.DMA,
            pltpu.SemaphoreType.DMA((C,)),
            pltpu.SemaphoreType.DMA((C,)),
            pltpu.SemaphoreType.DMA((C,)),
            pltpu.SemaphoreType.DMA((C,)),
        ],
        compiler_params=pltpu.CompilerParams(
            collective_id=0, vmem_limit_bytes=64 * 1024 * 1024),
    )(Oj, Wo)
    return out
